# baseline (speedup 1.0000x reference)
import functools

import jax
import jax.numpy as jnp
from jax import lax
from jax.experimental import pallas as pl
from jax.experimental.pallas import tpu as pltpu
from jax.experimental.pallas import tpu_sc as plsc

CH = 128
NBUF = 5


@functools.partial(jax.jit, static_argnums=())
def kernel(x, table):
    R, C = x.shape
    V, D = table.shape
    B = R * C
    RT = R // CH
    DG = D // 8

    info = plsc.get_sparse_core_info()
    NC, NS = info.num_cores, info.num_subcores
    NW = NC * NS

    n_total_ch = B // CH
    n_ch = n_total_ch // NW
    n_grp = n_ch // NBUF
    assert n_ch * NW == n_total_ch and n_grp * NBUF == n_ch
    assert R % CH == 0 and D % 8 == 0

    xq = jnp.swapaxes(x, 0, 1).reshape(n_total_ch, CH).astype(jnp.int32)

    mesh = plsc.VectorSubcoreMesh(core_axis_name="c", subcore_axis_name="s")

    @functools.partial(
        pl.kernel,
        mesh=mesh,
        compiler_params=pltpu.CompilerParams(
            use_tc_tiling_on_sc=False, needs_layout_passes=False
        ),
        out_type=jax.ShapeDtypeStruct((C, DG, RT, 8, CH), jnp.float32),
        scratch_types=[
            pltpu.VMEM((n_ch, CH), jnp.int32),
            *[pltpu.VMEM((CH, D), jnp.float32) for _ in range(NBUF)],
            *[pltpu.VMEM((DG, 8, CH), jnp.float32) for _ in range(NBUF)],
            pltpu.SemaphoreType.DMA((NBUF,)),
            pltpu.SemaphoreType.DMA((NBUF,)),
        ],
    )
    def gather_k(x_hbm, table_hbm, out_hbm, idx_v, *bufs):
        rows_b = bufs[:NBUF]
        tr_b = bufs[NBUF:2 * NBUF]
        gsem, wsem = bufs[2 * NBUF], bufs[2 * NBUF + 1]
        wid = lax.axis_index("s") * NC + lax.axis_index("c")
        row0 = wid * n_ch
        pltpu.sync_copy(x_hbm.at[pl.ds(row0, n_ch)], idx_v)
        iota = lax.iota(jnp.int32, 16)

        perms = [(iota + s) & 15 for s in range(16)]

        def transpose_chunk(rows_v, tr_v):
            def mloop(m, carry):
                lvec = iota + m * 16
                for q in range(D // 16):
                    for s in range(16):
                        d = perms[s] + q * 16
                        vals = plsc.load_gather(rows_v, [lvec, d])
                        plsc.store_scatter(tr_v, [d // 8, d % 8, lvec], vals)
                return carry

            lax.fori_loop(0, CH // 16, mloop, 0)

        def group(g, carry):
            c0 = g * NBUF
            gathers = []
            for b in range(NBUF):
                @pl.when(g > 0)
                def _drain(b=b):
                    pltpu.make_async_copy(
                        tr_b[b],
                        out_hbm.at[0, :, 0],
                        wsem.at[b],
                    ).wait()

                gathers.append(
                    pltpu.async_copy(
                        table_hbm.at[idx_v.at[c0 + b]], rows_b[b], gsem.at[b]
                    )
                )
            for b in range(NBUF):
                gathers[b].wait()
                transpose_chunk(rows_b[b], tr_b[b])
                m = row0 + c0 + b
                jj = m // RT
                cc = m % RT
                pltpu.async_copy(
                    tr_b[b],
                    out_hbm.at[jj, :, cc],
                    wsem.at[b],
                )
            return carry

        lax.fori_loop(0, n_grp, group, 0)
        for b in range(NBUF):
            pltpu.make_async_copy(
                tr_b[b],
                out_hbm.at[0, :, 0],
                wsem.at[b],
            ).wait()

    out5 = gather_k(xq, table)
    return out5.transpose(2, 4, 0, 1, 3).reshape(R, C, D)


# speedup vs baseline: 1.0047x; 1.0047x over previous
import functools

import jax
import jax.numpy as jnp
from jax import lax
from jax.experimental import pallas as pl
from jax.experimental.pallas import tpu as pltpu
from jax.experimental.pallas import tpu_sc as plsc

CH = 128
NBUF = 5


@functools.partial(jax.jit, static_argnums=())
def kernel(x, table):
    R, C = x.shape
    V, D = table.shape
    B = R * C
    RT = R // CH
    DG = D // 8

    info = plsc.get_sparse_core_info()
    NC, NS = info.num_cores, info.num_subcores
    NW = NC * NS

    n_total_ch = B // CH
    n_ch = n_total_ch // NW
    n_grp = n_ch // NBUF
    assert n_ch * NW == n_total_ch and n_grp * NBUF == n_ch
    assert R % CH == 0 and D % 8 == 0

    xq = jnp.swapaxes(x, 0, 1).reshape(n_total_ch, CH).astype(jnp.int32)

    mesh = plsc.VectorSubcoreMesh(core_axis_name="c", subcore_axis_name="s")

    @functools.partial(
        pl.kernel,
        mesh=mesh,
        compiler_params=pltpu.CompilerParams(
            use_tc_tiling_on_sc=False, needs_layout_passes=False
        ),
        out_type=jax.ShapeDtypeStruct((C, DG, RT, 8, CH), jnp.float32),
        scratch_types=[
            pltpu.VMEM((n_ch, CH), jnp.int32),
            *[pltpu.VMEM((CH, D), jnp.float32) for _ in range(NBUF)],
            *[pltpu.VMEM((DG, 8, CH), jnp.float32) for _ in range(NBUF)],
            pltpu.SemaphoreType.DMA((NBUF,)),
            pltpu.SemaphoreType.DMA((NBUF,)),
        ],
    )
    def gather_k(x_hbm, table_hbm, out_hbm, idx_v, *bufs):
        rows_b = bufs[:NBUF]
        tr_b = bufs[NBUF:2 * NBUF]
        gsem, wsem = bufs[2 * NBUF], bufs[2 * NBUF + 1]
        wid = lax.axis_index("s") * NC + lax.axis_index("c")
        row0 = wid * n_ch
        pltpu.sync_copy(x_hbm.at[pl.ds(row0, n_ch)], idx_v)
        iota = lax.iota(jnp.int32, 16)

        perms = [(iota + s) & 15 for s in range(16)]

        def transpose_chunk(rows_v, tr_v):
            def mloop(m, carry):
                lvec = iota + m * 16
                for q in range(D // 16):
                    for s in range(16):
                        d = perms[s] + q * 16
                        vals = plsc.load_gather(rows_v, [lvec, d])
                        plsc.store_scatter(tr_v, [d // 8, d % 8, lvec], vals)
                return carry

            lax.fori_loop(0, CH // 16, mloop, 0)

        def group(g, carry):
            c0 = g * NBUF
            gathers = []
            for b in range(NBUF):
                @pl.when(g > 0)
                def _drain(b=b):
                    for r in range(DG):
                        pltpu.make_async_copy(
                            tr_b[b].at[r],
                            out_hbm.at[0, r, 0],
                            wsem.at[b],
                        ).wait()

                gathers.append(
                    pltpu.async_copy(
                        table_hbm.at[idx_v.at[c0 + b]], rows_b[b], gsem.at[b]
                    )
                )
            for b in range(NBUF):
                gathers[b].wait()
                transpose_chunk(rows_b[b], tr_b[b])
                m = row0 + c0 + b
                jj = m // RT
                cc = m % RT
                for r in range(DG):
                    pltpu.async_copy(
                        tr_b[b].at[r],
                        out_hbm.at[jj, r, cc],
                        wsem.at[b],
                    )
            return carry

        lax.fori_loop(0, n_grp, group, 0)
        for b in range(NBUF):
            for r in range(DG):
                pltpu.make_async_copy(
                    tr_b[b].at[r],
                    out_hbm.at[0, r, 0],
                    wsem.at[b],
                ).wait()

    out5 = gather_k(xq, table)
    return out5.transpose(2, 4, 0, 1, 3).reshape(R, C, D)


# speedup vs baseline: 1.1932x; 1.1876x over previous
import functools

import jax
import jax.numpy as jnp
from jax import lax
from jax.experimental import pallas as pl
from jax.experimental.pallas import tpu as pltpu
from jax.experimental.pallas import tpu_sc as plsc

CH = 128
NBUF = 4


@functools.partial(jax.jit, static_argnums=())
def kernel(x, table):
    R, C = x.shape
    V, D = table.shape
    B = R * C
    RT = R // CH
    DG = D // 8

    info = plsc.get_sparse_core_info()
    NC, NS = info.num_cores, info.num_subcores
    NW = NC * NS

    n_total_ch = B // CH
    n_ch = n_total_ch // NW
    n_grp = n_ch // NBUF
    assert n_ch * NW == n_total_ch and n_grp * NBUF == n_ch
    assert R % CH == 0 and D % 8 == 0

    xq = jnp.swapaxes(x, 0, 1).reshape(n_total_ch, CH).astype(jnp.int32)

    mesh = plsc.VectorSubcoreMesh(core_axis_name="c", subcore_axis_name="s")

    @functools.partial(
        pl.kernel,
        mesh=mesh,
        compiler_params=pltpu.CompilerParams(
            use_tc_tiling_on_sc=False, needs_layout_passes=False
        ),
        out_type=jax.ShapeDtypeStruct((C, DG, RT, 8, CH), jnp.float32),
        scratch_types=[
            pltpu.VMEM((n_ch, CH), jnp.int32),
            *[pltpu.VMEM((CH, D), jnp.float32) for _ in range(NBUF)],
            *[pltpu.VMEM((DG, 8, CH), jnp.float32) for _ in range(NBUF)],
            pltpu.SemaphoreType.DMA((NBUF,)),
            pltpu.SemaphoreType.DMA((NBUF,)),
        ],
    )
    def gather_k(x_hbm, table_hbm, out_hbm, idx_v, *bufs):
        rows_b = bufs[:NBUF]
        tr_b = bufs[NBUF:2 * NBUF]
        gsem, wsem = bufs[2 * NBUF], bufs[2 * NBUF + 1]
        wid = lax.axis_index("s") * NC + lax.axis_index("c")
        row0 = wid * n_ch
        pltpu.sync_copy(x_hbm.at[pl.ds(row0, n_ch)], idx_v)
        iota = lax.iota(jnp.int32, 16)

        perms = [(iota + s) & 15 for s in range(16)]

        def transpose_chunk(rows_v, tr_v):
            def mqloop(mq, carry):
                m = mq // 4
                q = mq % 4
                lvec = iota + m * 16
                for s in range(16):
                    d = perms[s] + q * 16
                    vals = plsc.load_gather(rows_v, [lvec, d])
                    plsc.store_scatter(tr_v, [d // 8, d % 8, lvec], vals)
                return carry

            lax.fori_loop(0, (CH // 16) * (D // 16), mqloop, 0)

        def group(g, carry):
            c0 = g * NBUF
            gathers = []
            for b in range(NBUF):
                @pl.when(g > 0)
                def _drain(b=b):
                    for r in range(DG):
                        pltpu.make_async_copy(
                            tr_b[b].at[r],
                            out_hbm.at[0, r, 0],
                            wsem.at[b],
                        ).wait()

                gathers.append(
                    pltpu.async_copy(
                        table_hbm.at[idx_v.at[c0 + b]], rows_b[b], gsem.at[b]
                    )
                )
            for b in range(NBUF):
                gathers[b].wait()
                transpose_chunk(rows_b[b], tr_b[b])
                m = row0 + c0 + b
                jj = m // RT
                cc = m % RT
                for r in range(DG):
                    pltpu.async_copy(
                        tr_b[b].at[r],
                        out_hbm.at[jj, r, cc],
                        wsem.at[b],
                    )
            return carry

        lax.fori_loop(0, n_grp, group, 0)
        for b in range(NBUF):
            for r in range(DG):
                pltpu.make_async_copy(
                    tr_b[b].at[r],
                    out_hbm.at[0, r, 0],
                    wsem.at[b],
                ).wait()

    out5 = gather_k(xq, table)
    return out5.transpose(2, 4, 0, 1, 3).reshape(R, C, D)


# speedup vs baseline: 1.2613x; 1.0570x over previous
import functools

import jax
import jax.numpy as jnp
from jax import lax
from jax.experimental import pallas as pl
from jax.experimental.pallas import tpu as pltpu
from jax.experimental.pallas import tpu_sc as plsc

CH = 128
NBUF = 4


@functools.partial(jax.jit, static_argnums=())
def kernel(x, table):
    R, C = x.shape
    V, D = table.shape
    B = R * C
    RT = R // CH
    DG = D // 8

    info = plsc.get_sparse_core_info()
    NC, NS = info.num_cores, info.num_subcores
    NW = NC * NS

    n_total_ch = B // CH
    n_ch = n_total_ch // NW
    n_grp = n_ch // NBUF
    assert n_ch * NW == n_total_ch and n_grp * NBUF == n_ch
    assert R % CH == 0 and D % 8 == 0

    xq = jnp.swapaxes(x, 0, 1).reshape(n_total_ch, CH).astype(jnp.int32)

    mesh = plsc.VectorSubcoreMesh(core_axis_name="c", subcore_axis_name="s")

    @functools.partial(
        pl.kernel,
        mesh=mesh,
        compiler_params=pltpu.CompilerParams(
            use_tc_tiling_on_sc=False, needs_layout_passes=False
        ),
        out_type=jax.ShapeDtypeStruct((C, DG, RT, 8, CH), jnp.float32),
        scratch_types=[
            pltpu.VMEM((n_ch, CH), jnp.int32),
            *[pltpu.VMEM((CH, D), jnp.float32) for _ in range(NBUF)],
            *[pltpu.VMEM((DG, 8, CH), jnp.float32) for _ in range(NBUF)],
            pltpu.SemaphoreType.DMA((NBUF,)),
            pltpu.SemaphoreType.DMA((NBUF,)),
        ],
    )
    def gather_k(x_hbm, table_hbm, out_hbm, idx_v, *bufs):
        rows_b = bufs[:NBUF]
        tr_b = bufs[NBUF:2 * NBUF]
        gsem, wsem = bufs[2 * NBUF], bufs[2 * NBUF + 1]
        wid = lax.axis_index("s") * NC + lax.axis_index("c")
        row0 = wid * n_ch
        pltpu.sync_copy(x_hbm.at[pl.ds(row0, n_ch)], idx_v)
        iota = lax.iota(jnp.int32, 16)

        perms = [(iota + s) & 15 for s in range(16)]

        def transpose_chunk(rows_v, tr_v):
            @plsc.parallel_loop(0, (CH // 16) * (D // 16), unroll=2)
            def mqloop(mq):
                m = mq // 4
                q = mq % 4
                lvec = iota + m * 16
                for s in range(16):
                    d = perms[s] + q * 16
                    vals = plsc.load_gather(rows_v, [lvec, d])
                    plsc.store_scatter(tr_v, [d // 8, d % 8, lvec], vals)

        def group(g, carry):
            c0 = g * NBUF
            gathers = []
            for b in range(NBUF):
                @pl.when(g > 0)
                def _drain(b=b):
                    for r in range(DG):
                        pltpu.make_async_copy(
                            tr_b[b].at[r],
                            out_hbm.at[0, r, 0],
                            wsem.at[b],
                        ).wait()

                gathers.append(
                    pltpu.async_copy(
                        table_hbm.at[idx_v.at[c0 + b]], rows_b[b], gsem.at[b]
                    )
                )
            for b in range(NBUF):
                gathers[b].wait()
                transpose_chunk(rows_b[b], tr_b[b])
                m = row0 + c0 + b
                jj = m // RT
                cc = m % RT
                for r in range(DG):
                    pltpu.async_copy(
                        tr_b[b].at[r],
                        out_hbm.at[jj, r, cc],
                        wsem.at[b],
                    )
            return carry

        lax.fori_loop(0, n_grp, group, 0)
        for b in range(NBUF):
            for r in range(DG):
                pltpu.make_async_copy(
                    tr_b[b].at[r],
                    out_hbm.at[0, r, 0],
                    wsem.at[b],
                ).wait()

    out5 = gather_k(xq, table)
    return out5.transpose(2, 4, 0, 1, 3).reshape(R, C, D)


# speedup vs baseline: 1.3033x; 1.0333x over previous
import functools

import jax
import jax.numpy as jnp
from jax import lax
from jax.experimental import pallas as pl
from jax.experimental.pallas import tpu as pltpu
from jax.experimental.pallas import tpu_sc as plsc

CH = 128
NBUF = 4


@functools.partial(jax.jit, static_argnums=())
def kernel(x, table):
    R, C = x.shape
    V, D = table.shape
    B = R * C
    RT = R // CH
    DG = D // 8

    info = plsc.get_sparse_core_info()
    NC, NS = info.num_cores, info.num_subcores
    NW = NC * NS

    n_total_ch = B // CH
    n_ch = n_total_ch // NW
    n_grp = n_ch // NBUF
    assert n_ch * NW == n_total_ch and n_grp * NBUF == n_ch
    assert R % CH == 0 and D % 8 == 0

    xq = jnp.swapaxes(x, 0, 1).reshape(n_total_ch, CH).astype(jnp.int32)

    mesh = plsc.VectorSubcoreMesh(core_axis_name="c", subcore_axis_name="s")

    @functools.partial(
        pl.kernel,
        mesh=mesh,
        compiler_params=pltpu.CompilerParams(
            use_tc_tiling_on_sc=False, needs_layout_passes=False
        ),
        out_type=jax.ShapeDtypeStruct((C, DG, RT, 8, CH), jnp.float32),
        scratch_types=[
            pltpu.VMEM((n_ch, CH), jnp.int32),
            *[pltpu.VMEM((CH, D), jnp.float32) for _ in range(NBUF)],
            *[pltpu.VMEM((DG, 8, CH), jnp.float32) for _ in range(NBUF)],
            pltpu.SemaphoreType.DMA((NBUF,)),
            pltpu.SemaphoreType.DMA((NBUF,)),
        ],
    )
    def gather_k(x_hbm, table_hbm, out_hbm, idx_v, *bufs):
        rows_b = bufs[:NBUF]
        tr_b = bufs[NBUF:2 * NBUF]
        gsem, wsem = bufs[2 * NBUF], bufs[2 * NBUF + 1]
        wid = lax.axis_index("s") * NC + lax.axis_index("c")
        row0 = wid * n_ch
        pltpu.sync_copy(x_hbm.at[pl.ds(row0, n_ch)], idx_v)
        iota = lax.iota(jnp.int32, 16)

        perms = [(iota + s) & 15 for s in range(16)]

        def transpose_chunk(rows_v, tr_v):
            @plsc.parallel_loop(0, (CH // 16) * (D // 16), unroll=4)
            def mqloop(mq):
                m = mq // 4
                q = mq % 4
                lvec = iota + m * 16
                for s in range(16):
                    d = perms[s] + q * 16
                    vals = plsc.load_gather(rows_v, [lvec, d])
                    plsc.store_scatter(tr_v, [d // 8, d % 8, lvec], vals)

        def group(g, carry):
            c0 = g * NBUF
            gathers = []
            for b in range(NBUF):
                @pl.when(g > 0)
                def _drain(b=b):
                    for r in range(DG):
                        pltpu.make_async_copy(
                            tr_b[b].at[r],
                            out_hbm.at[0, r, 0],
                            wsem.at[b],
                        ).wait()

                gathers.append(
                    pltpu.async_copy(
                        table_hbm.at[idx_v.at[c0 + b]], rows_b[b], gsem.at[b]
                    )
                )
            for b in range(NBUF):
                gathers[b].wait()
                transpose_chunk(rows_b[b], tr_b[b])
                m = row0 + c0 + b
                jj = m // RT
                cc = m % RT
                for r in range(DG):
                    pltpu.async_copy(
                        tr_b[b].at[r],
                        out_hbm.at[jj, r, cc],
                        wsem.at[b],
                    )
            return carry

        lax.fori_loop(0, n_grp, group, 0)
        for b in range(NBUF):
            for r in range(DG):
                pltpu.make_async_copy(
                    tr_b[b].at[r],
                    out_hbm.at[0, r, 0],
                    wsem.at[b],
                ).wait()

    out5 = gather_k(xq, table)
    return out5.transpose(2, 4, 0, 1, 3).reshape(R, C, D)


# speedup vs baseline: 1.4582x; 1.1188x over previous
import functools

import jax
import jax.numpy as jnp
from jax import lax
from jax.experimental import pallas as pl
from jax.experimental.pallas import tpu as pltpu
from jax.experimental.pallas import tpu_sc as plsc

CH = 128
NBUF = 4


@functools.partial(jax.jit, static_argnums=())
def kernel(x, table):
    R, C = x.shape
    V, D = table.shape
    B = R * C
    RT = R // CH
    DG = D // 8

    info = plsc.get_sparse_core_info()
    NC, NS = info.num_cores, info.num_subcores
    NW = NC * NS

    n_total_ch = B // CH
    n_ch = n_total_ch // NW
    n_grp = n_ch // NBUF
    assert n_ch * NW == n_total_ch and n_grp * NBUF == n_ch
    assert R % CH == 0 and D % 8 == 0

    xq = jnp.swapaxes(x, 0, 1).reshape(n_total_ch, CH).astype(jnp.int32)

    mesh = plsc.VectorSubcoreMesh(core_axis_name="c", subcore_axis_name="s")

    @functools.partial(
        pl.kernel,
        mesh=mesh,
        compiler_params=pltpu.CompilerParams(
            use_tc_tiling_on_sc=False, needs_layout_passes=False
        ),
        out_type=jax.ShapeDtypeStruct((C, DG, RT, 8, CH), jnp.float32),
        scratch_types=[
            pltpu.VMEM((n_ch, CH), jnp.int32),
            *[pltpu.VMEM((CH, D), jnp.float32) for _ in range(NBUF)],
            *[pltpu.VMEM((DG, 8, CH), jnp.float32) for _ in range(NBUF)],
            pltpu.SemaphoreType.DMA((NBUF,)),
            pltpu.SemaphoreType.DMA((NBUF,)),
        ],
    )
    def gather_k(x_hbm, table_hbm, out_hbm, idx_v, *bufs):
        rows_b = bufs[:NBUF]
        tr_b = bufs[NBUF:2 * NBUF]
        gsem, wsem = bufs[2 * NBUF], bufs[2 * NBUF + 1]
        wid = lax.axis_index("s") * NC + lax.axis_index("c")
        row0 = wid * n_ch
        pltpu.sync_copy(x_hbm.at[pl.ds(row0, n_ch)], idx_v)
        iota = lax.iota(jnp.int32, 16)

        perms = [(iota + s) & 15 for s in range(16)]

        def transpose_chunk(rows_v, tr_v):
            @plsc.parallel_loop(0, (CH // 16) * (D // 16), unroll=8)
            def mqloop(mq):
                m = mq // 4
                q = mq % 4
                lvec = iota + m * 16
                for s in range(16):
                    d = perms[s] + q * 16
                    vals = plsc.load_gather(rows_v, [lvec, d])
                    plsc.store_scatter(tr_v, [d // 8, d % 8, lvec], vals)

        def group(g, carry):
            c0 = g * NBUF
            gathers = []
            for b in range(NBUF):
                @pl.when(g > 0)
                def _drain(b=b):
                    for r in range(DG):
                        pltpu.make_async_copy(
                            tr_b[b].at[r],
                            out_hbm.at[0, r, 0],
                            wsem.at[b],
                        ).wait()

                gathers.append(
                    pltpu.async_copy(
                        table_hbm.at[idx_v.at[c0 + b]], rows_b[b], gsem.at[b]
                    )
                )
            for b in range(NBUF):
                gathers[b].wait()
                transpose_chunk(rows_b[b], tr_b[b])
                m = row0 + c0 + b
                jj = m // RT
                cc = m % RT
                for r in range(DG):
                    pltpu.async_copy(
                        tr_b[b].at[r],
                        out_hbm.at[jj, r, cc],
                        wsem.at[b],
                    )
            return carry

        lax.fori_loop(0, n_grp, group, 0)
        for b in range(NBUF):
            for r in range(DG):
                pltpu.make_async_copy(
                    tr_b[b].at[r],
                    out_hbm.at[0, r, 0],
                    wsem.at[b],
                ).wait()

    out5 = gather_k(xq, table)
    return out5.transpose(2, 4, 0, 1, 3).reshape(R, C, D)
